# Initial kernel scaffold; baseline (speedup 1.0000x reference)
import jax
import jax.numpy as jnp
from jax.experimental import pallas as pl


def kernel(x, codebook):
    raise NotImplementedError("write your pallas kernel here")


# speedup vs baseline: 2.0903x; 2.0903x over previous
import functools

import jax
import jax.numpy as jnp
from jax.experimental import pallas as pl
from jax.experimental.pallas import tpu as pltpu

NB = 1024
CD = 256
BT = 512
NTOK = 16 * 576
NBLK = NTOK // BT


def _vq_kernel(x_ref, cbt_ref, cb_ref, out_ref, loss_ref, perp_ref,
               counts_ref, lsum_ref):
    i = pl.program_id(0)
    x = x_ref[...]
    cbt = cbt_ref[...]

    mm = jnp.dot(x, cbt, preferred_element_type=jnp.float32)
    x2 = jnp.sum(x * x, axis=1, keepdims=True)
    c2 = jnp.sum(cbt * cbt, axis=0, keepdims=True)
    dist = (x2 - 2.0 * mm) + c2

    neg = -dist
    m = jnp.max(neg, axis=1, keepdims=True)
    codes = jax.lax.broadcasted_iota(jnp.int32, neg.shape, 1)
    idx = jnp.min(jnp.where(neg == m, codes, NB), axis=1, keepdims=True)
    onehot = (codes == idx).astype(jnp.float32)

    x_d = jax.lax.dot_general(onehot, cb_ref[...], (((1,), (0,)), ((), ())),
                              precision=jax.lax.Precision.HIGHEST,
                              preferred_element_type=jnp.float32)
    out_ref[...] = x_d

    diff = x - x_d
    blk_loss = jnp.sum(diff * diff)
    blk_counts = jnp.sum(onehot, axis=0, keepdims=True)

    @pl.when(i == 0)
    def _init():
        counts_ref[...] = blk_counts
        lsum_ref[0, 0] = blk_loss

    @pl.when(i > 0)
    def _acc():
        counts_ref[...] += blk_counts
        lsum_ref[0, 0] += blk_loss

    @pl.when(i == NBLK - 1)
    def _fin():
        counts = counts_ref[...]
        prob = counts / jnp.sum(counts)
        perp = jnp.exp(-jnp.sum(prob * jnp.log(prob + 1e-07)))
        perp_ref[...] = perp.reshape(1, 1)
        loss_ref[...] = (lsum_ref[0, 0] / jnp.float32(NTOK * CD)).reshape(1, 1)


@functools.partial(jax.jit, static_argnames=())
def kernel(x, codebook):
    N, T, C = x.shape
    xf = x.reshape(-1, C)
    cbt = codebook.T

    out, loss, perp = pl.pallas_call(
        _vq_kernel,
        grid=(NBLK,),
        in_specs=[
            pl.BlockSpec((BT, CD), lambda i: (i, 0)),
            pl.BlockSpec((CD, NB), lambda i: (0, 0)),
            pl.BlockSpec((NB, CD), lambda i: (0, 0)),
        ],
        out_specs=[
            pl.BlockSpec((BT, CD), lambda i: (i, 0)),
            pl.BlockSpec((1, 1), lambda i: (0, 0)),
            pl.BlockSpec((1, 1), lambda i: (0, 0)),
        ],
        out_shape=[
            jax.ShapeDtypeStruct((NTOK, CD), jnp.float32),
            jax.ShapeDtypeStruct((1, 1), jnp.float32),
            jax.ShapeDtypeStruct((1, 1), jnp.float32),
        ],
        scratch_shapes=[
            pltpu.VMEM((1, NB), jnp.float32),
            pltpu.SMEM((1, 1), jnp.float32),
        ],
    )(xf, cbt, codebook)

    return (out.reshape(N, T, C), loss[0, 0], perp[0, 0])


# speedup vs baseline: 3.3398x; 1.5978x over previous
import functools

import jax
import jax.numpy as jnp
from jax.experimental import pallas as pl
from jax.experimental.pallas import tpu as pltpu

NB = 1024
CD = 256
BT = 512
NTOK = 16 * 576
NBLK = NTOK // BT


def _vq_kernel(x_ref, cbt_ref, cb_ref, out_ref, loss_ref, perp_ref,
               counts_ref, lsum_ref, c2_ref):
    i = pl.program_id(0)
    x = x_ref[...]
    cbt = cbt_ref[...]

    @pl.when(i == 0)
    def _c2():
        c2_ref[...] = jnp.sum(cbt * cbt, axis=0, keepdims=True)

    mm = jnp.dot(x, cbt, preferred_element_type=jnp.float32)
    x2 = jnp.sum(x * x, axis=1, keepdims=True)
    dist = (x2 - 2.0 * mm) + c2_ref[...]

    neg = -dist
    m = jnp.max(neg, axis=1, keepdims=True)
    codes = jax.lax.broadcasted_iota(jnp.int32, neg.shape, 1)
    idx = jnp.min(jnp.where(neg == m, codes, NB), axis=1, keepdims=True)
    onehot = (codes == idx).astype(jnp.float32)

    x_d = jax.lax.dot_general(onehot, cb_ref[...], (((1,), (0,)), ((), ())),
                              preferred_element_type=jnp.float32)
    out_ref[...] = x_d

    blk_loss = -jnp.sum(m)
    blk_counts = jnp.sum(onehot, axis=0, keepdims=True)

    @pl.when(i == 0)
    def _init():
        counts_ref[...] = blk_counts
        lsum_ref[0, 0] = blk_loss

    @pl.when(i > 0)
    def _acc():
        counts_ref[...] += blk_counts
        lsum_ref[0, 0] += blk_loss

    @pl.when(i == NBLK - 1)
    def _fin():
        counts = counts_ref[...]
        prob = counts / jnp.sum(counts)
        perp = jnp.exp(-jnp.sum(prob * jnp.log(prob + 1e-07)))
        perp_ref[...] = perp.reshape(1, 1)
        loss_ref[...] = (lsum_ref[0, 0] / jnp.float32(NTOK * CD)).reshape(1, 1)


@functools.partial(jax.jit, static_argnames=())
def kernel(x, codebook):
    N, T, C = x.shape
    xf = x.reshape(-1, C)
    cbt = codebook.T

    out, loss, perp = pl.pallas_call(
        _vq_kernel,
        grid=(NBLK,),
        in_specs=[
            pl.BlockSpec((BT, CD), lambda i: (i, 0)),
            pl.BlockSpec((CD, NB), lambda i: (0, 0)),
            pl.BlockSpec((NB, CD), lambda i: (0, 0)),
        ],
        out_specs=[
            pl.BlockSpec((BT, CD), lambda i: (i, 0)),
            pl.BlockSpec((1, 1), lambda i: (0, 0)),
            pl.BlockSpec((1, 1), lambda i: (0, 0)),
        ],
        out_shape=[
            jax.ShapeDtypeStruct((NTOK, CD), jnp.float32),
            jax.ShapeDtypeStruct((1, 1), jnp.float32),
            jax.ShapeDtypeStruct((1, 1), jnp.float32),
        ],
        scratch_shapes=[
            pltpu.VMEM((1, NB), jnp.float32),
            pltpu.SMEM((1, 1), jnp.float32),
            pltpu.VMEM((1, NB), jnp.float32),
        ],
    )(xf, cbt, codebook)

    return (out.reshape(N, T, C), loss[0, 0], perp[0, 0])


# speedup vs baseline: 3.4505x; 1.0331x over previous
import functools

import jax
import jax.numpy as jnp
from jax.experimental import pallas as pl
from jax.experimental.pallas import tpu as pltpu

NB = 1024
CD = 256
BT = 512
NTOK = 16 * 576
NBLK = NTOK // BT


def _vq_kernel(x_ref, cbt_ref, cb_ref, out_ref, loss_ref, perp_ref,
               counts_ref, lsum_ref, c2_ref):
    i = pl.program_id(0)
    x = x_ref[...]
    cbt = cbt_ref[...]

    @pl.when(i == 0)
    def _c2():
        c2_ref[...] = jnp.sum(cbt * cbt, axis=0, keepdims=True)

    mm = jnp.dot(x, cbt, preferred_element_type=jnp.float32)
    x2 = jnp.sum(x * x, axis=1, keepdims=True)
    dist = (x2 - 2.0 * mm) + c2_ref[...]

    mn = jnp.min(dist, axis=1, keepdims=True)
    codes = jax.lax.broadcasted_iota(jnp.int32, dist.shape, 1)
    idx = jnp.min(jnp.where(dist == mn, codes, NB), axis=1, keepdims=True)
    onehot = (codes == idx).astype(jnp.float32)

    x_d = jax.lax.dot_general(onehot, cb_ref[...], (((1,), (0,)), ((), ())),
                              preferred_element_type=jnp.float32)
    out_ref[...] = x_d

    blk_loss = jnp.sum(mn)
    blk_counts = jnp.sum(onehot, axis=0, keepdims=True)

    @pl.when(i == 0)
    def _init():
        counts_ref[...] = blk_counts
        lsum_ref[0, 0] = blk_loss

    @pl.when(i > 0)
    def _acc():
        counts_ref[...] += blk_counts
        lsum_ref[0, 0] += blk_loss

    @pl.when(i == NBLK - 1)
    def _fin():
        counts = counts_ref[...]
        prob = counts / jnp.sum(counts)
        perp = jnp.exp(-jnp.sum(prob * jnp.log(prob + 1e-07)))
        perp_ref[...] = perp.reshape(1, 1)
        loss_ref[...] = (lsum_ref[0, 0] / jnp.float32(NTOK * CD)).reshape(1, 1)


@functools.partial(jax.jit, static_argnames=())
def kernel(x, codebook):
    N, T, C = x.shape
    xf = x.reshape(-1, C)
    cbt = codebook.T

    out, loss, perp = pl.pallas_call(
        _vq_kernel,
        grid=(NBLK,),
        in_specs=[
            pl.BlockSpec((BT, CD), lambda i: (i, 0)),
            pl.BlockSpec((CD, NB), lambda i: (0, 0)),
            pl.BlockSpec((NB, CD), lambda i: (0, 0)),
        ],
        out_specs=[
            pl.BlockSpec((BT, CD), lambda i: (i, 0)),
            pl.BlockSpec((1, 1), lambda i: (0, 0)),
            pl.BlockSpec((1, 1), lambda i: (0, 0)),
        ],
        out_shape=[
            jax.ShapeDtypeStruct((NTOK, CD), jnp.float32),
            jax.ShapeDtypeStruct((1, 1), jnp.float32),
            jax.ShapeDtypeStruct((1, 1), jnp.float32),
        ],
        scratch_shapes=[
            pltpu.VMEM((1, NB), jnp.float32),
            pltpu.SMEM((1, 1), jnp.float32),
            pltpu.VMEM((1, NB), jnp.float32),
        ],
    )(xf, cbt, codebook)

    return (out.reshape(N, T, C), loss[0, 0], perp[0, 0])
